# Initial kernel scaffold; baseline (speedup 1.0000x reference)
import jax
import jax.numpy as jnp
from jax.experimental import pallas as pl


def kernel(x, n_id, edge_index, W_sage, b_sage, W_cls, b_cls):
    raise NotImplementedError("write your pallas kernel here")


# speedup vs baseline: 1.2773x; 1.2773x over previous
import functools

import jax
import jax.numpy as jnp
from jax import lax
from jax.experimental import pallas as pl
from jax.experimental.pallas import tpu as pltpu

N_NODES = 10000
D_IN = 128
N_CLS = 3


def _project_body(xs_ref, w_sage_ref, w_cls_ref, z4_ref):
    w_fused = jnp.dot(w_sage_ref[...], w_cls_ref[...],
                      preferred_element_type=jnp.float32)
    z = jnp.dot(xs_ref[...], w_fused, preferred_element_type=jnp.float32)
    z4_ref[:, 0:N_CLS] = z
    z4_ref[:, N_CLS:4] = jnp.ones((z.shape[0], 1), jnp.float32)


def _project(xs, w_sage, w_cls):
    return pl.pallas_call(
        _project_body,
        out_shape=jax.ShapeDtypeStruct((N_NODES, 4), jnp.float32),
    )(xs, w_sage, w_cls)


def _finalize_body(acc_ref, w_cls_ref, b_sage_ref, b_cls_ref, out_ref):
    acc = acc_ref[...]
    cnt = jnp.clip(acc[:, 3:4], 1.0, None)
    b_eff = jnp.dot(b_sage_ref[...].reshape(1, -1), w_cls_ref[...],
                    preferred_element_type=jnp.float32) + b_cls_ref[...].reshape(1, -1)
    s = acc[:, 0:N_CLS] / cnt + b_eff
    m = jnp.max(s, axis=1, keepdims=True)
    lse = jnp.log(jnp.sum(jnp.exp(s - m), axis=1, keepdims=True)) + m
    out_ref[...] = s - lse


def _finalize(acc, w_cls, b_sage, b_cls):
    return pl.pallas_call(
        _finalize_body,
        out_shape=jax.ShapeDtypeStruct((N_NODES, N_CLS), jnp.float32),
    )(acc, w_cls, b_sage, b_cls)


def kernel(x, n_id, edge_index, W_sage, b_sage, W_cls, b_cls):
    xs = jnp.take(x, n_id, axis=0)
    z4 = _project(xs, W_sage, W_cls)
    src = edge_index[0]
    dst = edge_index[1]
    msgs = jnp.take(z4, src, axis=0)
    acc = jax.ops.segment_sum(msgs, dst, num_segments=N_NODES)
    return _finalize(acc, W_cls, b_sage, b_cls)


# speedup vs baseline: 17.5170x; 13.7142x over previous
import functools

import jax
import jax.numpy as jnp
from jax import lax
from jax.experimental import pallas as pl
from jax.experimental.pallas import tpu as pltpu
from jax.experimental.pallas import tpu_sc as plsc

N_NODES = 10000
D_IN = 128
N_CLS = 3

NC = 2
NS = 16
NW = NC * NS

N_PAD = 10240
ROWS_W = N_PAD // NW
E_PAD = 327680
CHUNKS_W = 80
CHUNK = 128
EDGES_W = CHUNKS_W * CHUNK
ROWS_OUT = N_PAD // NS

_sc_mesh = plsc.VectorSubcoreMesh(core_axis_name="c", subcore_axis_name="s")


@functools.partial(
    pl.kernel,
    out_type=jax.ShapeDtypeStruct((N_PAD, D_IN), jnp.float32),
    mesh=_sc_mesh,
    scratch_types=[
        pltpu.VMEM((4, ROWS_W // 4), jnp.int32),
        pltpu.VMEM((ROWS_W, D_IN), jnp.float32),
        pltpu.SemaphoreType.DMA,
    ],
    compiler_params=pltpu.CompilerParams(use_tc_tiling_on_sc=False),
)
def _gather_rows(x_hbm, nid_hbm, xs_hbm, idx_v, rows_v, sem):
    wid = lax.axis_index("c") * NS + lax.axis_index("s")
    pltpu.sync_copy(nid_hbm.at[wid], idx_v)
    q = ROWS_W // 4
    handles = [
        pltpu.async_copy(x_hbm.at[idx_v.at[i]], rows_v.at[pl.ds(i * q, q)], sem)
        for i in range(4)
    ]
    for h in handles:
        h.wait()
    pltpu.sync_copy(rows_v, xs_hbm.at[pl.ds(wid * ROWS_W, ROWS_W)])


def _project_body(xs_ref, w_sage_ref, w_cls_ref, z4_ref):
    w_fused = jnp.dot(w_sage_ref[...], w_cls_ref[...],
                      preferred_element_type=jnp.float32)
    z = jnp.dot(xs_ref[...], w_fused, preferred_element_type=jnp.float32)
    z4_ref[:, 0:N_CLS] = z
    z4_ref[:, N_CLS:4] = jnp.ones((z.shape[0], 1), jnp.float32)
    z4_ref[:, 4:8] = jnp.zeros((z.shape[0], 4), jnp.float32)


def _project(xs, w_sage, w_cls):
    return pl.pallas_call(
        _project_body,
        out_shape=jax.ShapeDtypeStruct((N_PAD, 8), jnp.float32),
    )(xs, w_sage, w_cls)


@functools.partial(
    pl.kernel,
    out_type=jax.ShapeDtypeStruct((NC, N_PAD, 8), jnp.float32),
    mesh=_sc_mesh,
    scratch_types=[
        pltpu.VMEM((CHUNKS_W, CHUNK), jnp.int32),
        pltpu.VMEM((CHUNKS_W, CHUNK), jnp.int32),
        pltpu.VMEM((EDGES_W, 8), jnp.float32),
        pltpu.VMEM_SHARED((N_PAD, 8), jnp.float32),
        pltpu.SemaphoreType.DMA,
        pltpu.SemaphoreType.DMA,
    ],
    compiler_params=pltpu.CompilerParams(use_tc_tiling_on_sc=False),
)
def _edge_aggregate(src_hbm, dst_hbm, zeros_hbm, z4_hbm, acc_hbm,
                    src_v, dst_v, upd_v, acc_sh, gsem, ssem):
    cid = lax.axis_index("c")
    sid = lax.axis_index("s")
    wid = cid * NS + sid

    pltpu.sync_copy(zeros_hbm.at[pl.ds(sid * ROWS_OUT, ROWS_OUT)],
                    acc_sh.at[pl.ds(sid * ROWS_OUT, ROWS_OUT)])
    pltpu.sync_copy(src_hbm.at[wid], src_v)
    pltpu.sync_copy(dst_hbm.at[wid], dst_v)
    plsc.subcore_barrier()

    k = 16

    def block(o, _):
        base = o * k
        gh = [
            pltpu.async_copy(
                z4_hbm.at[src_v.at[base + i]],
                upd_v.at[pl.ds((base + i) * CHUNK, CHUNK)], gsem)
            for i in range(k)
        ]
        for h in gh:
            h.wait()
        sh = [
            pltpu.async_copy(
                upd_v.at[pl.ds((base + i) * CHUNK, CHUNK)],
                acc_sh.at[dst_v.at[base + i]], ssem, add=True)
            for i in range(k)
        ]
        for h in sh:
            h.wait()
        return _

    lax.fori_loop(0, CHUNKS_W // k, block, 0, unroll=False)

    plsc.subcore_barrier()
    pltpu.sync_copy(acc_sh.at[pl.ds(sid * ROWS_OUT, ROWS_OUT)],
                    acc_hbm.at[cid].at[pl.ds(sid * ROWS_OUT, ROWS_OUT)])


def _finalize_body(acc_ref, w_cls_ref, b_sage_ref, b_cls_ref, out_ref):
    acc = acc_ref[0] + acc_ref[1]
    cnt = jnp.clip(acc[:N_NODES, 3:4], 1.0, None)
    b_eff = jnp.dot(b_sage_ref[...].reshape(1, -1), w_cls_ref[...],
                    preferred_element_type=jnp.float32) + b_cls_ref[...].reshape(1, -1)
    s = acc[:N_NODES, 0:N_CLS] / cnt + b_eff
    m = jnp.max(s, axis=1, keepdims=True)
    lse = jnp.log(jnp.sum(jnp.exp(s - m), axis=1, keepdims=True)) + m
    out_ref[...] = s - lse


def _finalize(acc, w_cls, b_sage, b_cls):
    return pl.pallas_call(
        _finalize_body,
        out_shape=jax.ShapeDtypeStruct((N_NODES, N_CLS), jnp.float32),
    )(acc, w_cls, b_sage, b_cls)


def kernel(x, n_id, edge_index, W_sage, b_sage, W_cls, b_cls):
    E = edge_index.shape[1]
    nid_pad = jnp.concatenate(
        [n_id, jnp.zeros((N_PAD - N_NODES,), jnp.int32)]).reshape(NW, 4, ROWS_W // 4)
    src_pad = jnp.concatenate(
        [edge_index[0], jnp.zeros((E_PAD - E,), jnp.int32)]).reshape(NW, CHUNKS_W, CHUNK)
    dst_pad = jnp.concatenate(
        [edge_index[1], jnp.full((E_PAD - E,), N_PAD - 1, jnp.int32)]).reshape(NW, CHUNKS_W, CHUNK)
    zeros_init = jnp.zeros((N_PAD, 8), jnp.float32)

    xs = _gather_rows(x, nid_pad)
    z4 = _project(xs, W_sage, W_cls)
    acc = _edge_aggregate(src_pad, dst_pad, zeros_init, z4)
    return _finalize(acc, W_cls, b_sage, b_cls)


# speedup vs baseline: 18.1637x; 1.0369x over previous
import functools

import jax
import jax.numpy as jnp
from jax import lax
from jax.experimental import pallas as pl
from jax.experimental.pallas import tpu as pltpu
from jax.experimental.pallas import tpu_sc as plsc

N_NODES = 10000
D_IN = 128
N_CLS = 3

NC = 2
NS = 16
NW = NC * NS

N_PAD = 10240
ROWS_W = N_PAD // NW
E_PAD = 327680
CHUNKS_W = 80
CHUNK = 128
EDGES_W = CHUNKS_W * CHUNK
ROWS_OUT = N_PAD // NS

_sc_mesh = plsc.VectorSubcoreMesh(core_axis_name="c", subcore_axis_name="s")


@functools.partial(
    pl.kernel,
    out_type=jax.ShapeDtypeStruct((N_PAD, D_IN), jnp.float32),
    mesh=_sc_mesh,
    scratch_types=[
        pltpu.VMEM((4, ROWS_W // 4), jnp.int32),
        pltpu.VMEM((ROWS_W, D_IN), jnp.float32),
        pltpu.SemaphoreType.DMA,
    ],
    compiler_params=pltpu.CompilerParams(use_tc_tiling_on_sc=False),
)
def _gather_rows(x_hbm, nid_hbm, xs_hbm, idx_v, rows_v, sem):
    wid = lax.axis_index("c") * NS + lax.axis_index("s")
    pltpu.sync_copy(nid_hbm.at[wid], idx_v)
    q = ROWS_W // 4
    handles = [
        pltpu.async_copy(x_hbm.at[idx_v.at[i]], rows_v.at[pl.ds(i * q, q)], sem)
        for i in range(4)
    ]
    for h in handles:
        h.wait()
    pltpu.sync_copy(rows_v, xs_hbm.at[pl.ds(wid * ROWS_W, ROWS_W)])


def _project_body(xs_ref, w_sage_ref, w_cls_ref, z4_ref):
    w_fused = jnp.dot(w_sage_ref[...], w_cls_ref[...],
                      preferred_element_type=jnp.float32)
    z = jnp.dot(xs_ref[...], w_fused, preferred_element_type=jnp.float32)
    z4_ref[:, 0:N_CLS] = z
    z4_ref[:, N_CLS:4] = jnp.ones((z.shape[0], 1), jnp.float32)
    z4_ref[:, 4:8] = jnp.zeros((z.shape[0], 4), jnp.float32)


def _project(xs, w_sage, w_cls):
    return pl.pallas_call(
        _project_body,
        out_shape=jax.ShapeDtypeStruct((N_PAD, 8), jnp.float32),
    )(xs, w_sage, w_cls)


@functools.partial(
    pl.kernel,
    out_type=jax.ShapeDtypeStruct((NC, N_PAD, 8), jnp.float32),
    mesh=_sc_mesh,
    scratch_types=[
        pltpu.VMEM((CHUNKS_W, CHUNK), jnp.int32),
        pltpu.VMEM((CHUNKS_W, CHUNK), jnp.int32),
        pltpu.VMEM((EDGES_W, 8), jnp.float32),
        pltpu.VMEM_SHARED((N_PAD, 8), jnp.float32),
        pltpu.SemaphoreType.DMA,
        pltpu.SemaphoreType.DMA,
        pltpu.SemaphoreType.DMA,
    ],
    compiler_params=pltpu.CompilerParams(use_tc_tiling_on_sc=False),
)
def _edge_aggregate(src_hbm, dst_hbm, zeros_hbm, z4_hbm, acc_hbm,
                    src_v, dst_v, upd_v, acc_sh, gsem0, gsem1, ssem):
    cid = lax.axis_index("c")
    sid = lax.axis_index("s")
    wid = cid * NS + sid

    pltpu.sync_copy(zeros_hbm.at[pl.ds(sid * ROWS_OUT, ROWS_OUT)],
                    acc_sh.at[pl.ds(sid * ROWS_OUT, ROWS_OUT)])
    pltpu.sync_copy(src_hbm.at[wid], src_v)
    pltpu.sync_copy(dst_hbm.at[wid], dst_v)
    plsc.subcore_barrier()

    K = 8
    NB = CHUNKS_W // K
    gsems = (gsem0, gsem1)

    def fire_gathers(b):
        for i in range(K):
            j = b * K + i
            pltpu.async_copy(z4_hbm.at[src_v.at[j]],
                             upd_v.at[pl.ds(j * CHUNK, CHUNK)], gsems[b % 2])

    fire_gathers(0)
    for b in range(NB):
        if b + 1 < NB:
            fire_gathers(b + 1)
        for i in range(K):
            j = b * K + i
            pltpu.make_async_copy(z4_hbm.at[src_v.at[j]],
                                  upd_v.at[pl.ds(j * CHUNK, CHUNK)],
                                  gsems[b % 2]).wait()
        for i in range(K):
            j = b * K + i
            pltpu.async_copy(upd_v.at[pl.ds(j * CHUNK, CHUNK)],
                             acc_sh.at[dst_v.at[j]], ssem, add=True)
    for j in range(CHUNKS_W):
        pltpu.make_async_copy(upd_v.at[pl.ds(j * CHUNK, CHUNK)],
                              acc_sh.at[dst_v.at[j]], ssem).wait()

    plsc.subcore_barrier()
    pltpu.sync_copy(acc_sh.at[pl.ds(sid * ROWS_OUT, ROWS_OUT)],
                    acc_hbm.at[cid].at[pl.ds(sid * ROWS_OUT, ROWS_OUT)])


def _finalize_body(acc_ref, w_cls_ref, b_sage_ref, b_cls_ref, out_ref):
    acc = acc_ref[0] + acc_ref[1]
    cnt = jnp.clip(acc[:N_NODES, 3:4], 1.0, None)
    b_eff = jnp.dot(b_sage_ref[...].reshape(1, -1), w_cls_ref[...],
                    preferred_element_type=jnp.float32) + b_cls_ref[...].reshape(1, -1)
    s = acc[:N_NODES, 0:N_CLS] / cnt + b_eff
    m = jnp.max(s, axis=1, keepdims=True)
    lse = jnp.log(jnp.sum(jnp.exp(s - m), axis=1, keepdims=True)) + m
    out_ref[...] = s - lse


def _finalize(acc, w_cls, b_sage, b_cls):
    return pl.pallas_call(
        _finalize_body,
        out_shape=jax.ShapeDtypeStruct((N_NODES, N_CLS), jnp.float32),
    )(acc, w_cls, b_sage, b_cls)


def kernel(x, n_id, edge_index, W_sage, b_sage, W_cls, b_cls):
    E = edge_index.shape[1]
    nid_pad = jnp.concatenate(
        [n_id, jnp.zeros((N_PAD - N_NODES,), jnp.int32)]).reshape(NW, 4, ROWS_W // 4)
    src_pad = jnp.concatenate(
        [edge_index[0], jnp.zeros((E_PAD - E,), jnp.int32)]).reshape(NW, CHUNKS_W, CHUNK)
    dst_pad = jnp.concatenate(
        [edge_index[1], jnp.full((E_PAD - E,), N_PAD - 1, jnp.int32)]).reshape(NW, CHUNKS_W, CHUNK)
    zeros_init = jnp.zeros((N_PAD, 8), jnp.float32)

    xs = _gather_rows(x, nid_pad)
    z4 = _project(xs, W_sage, W_cls)
    acc = _edge_aggregate(src_pad, dst_pad, zeros_init, z4)
    return _finalize(acc, W_cls, b_sage, b_cls)


# speedup vs baseline: 18.6922x; 1.0291x over previous
import functools

import jax
import jax.numpy as jnp
from jax import lax
from jax.experimental import pallas as pl
from jax.experimental.pallas import tpu as pltpu
from jax.experimental.pallas import tpu_sc as plsc

N_NODES = 10000
D_IN = 128
N_CLS = 3

NC = 2
NS = 16
NW = NC * NS

N_PAD = 10240
ROWS_W = N_PAD // NW
E_PAD = 327680
CHUNKS_W = 10
CHUNK = 1024
EDGES_W = CHUNKS_W * CHUNK
ROWS_OUT = N_PAD // NS

_sc_mesh = plsc.VectorSubcoreMesh(core_axis_name="c", subcore_axis_name="s")


@functools.partial(
    pl.kernel,
    out_type=jax.ShapeDtypeStruct((N_PAD, D_IN), jnp.float32),
    mesh=_sc_mesh,
    scratch_types=[
        pltpu.VMEM((4, ROWS_W // 4), jnp.int32),
        pltpu.VMEM((ROWS_W, D_IN), jnp.float32),
        pltpu.SemaphoreType.DMA,
    ],
    compiler_params=pltpu.CompilerParams(use_tc_tiling_on_sc=False),
)
def _gather_rows(x_hbm, nid_hbm, xs_hbm, idx_v, rows_v, sem):
    wid = lax.axis_index("c") * NS + lax.axis_index("s")
    pltpu.sync_copy(nid_hbm.at[wid], idx_v)
    q = ROWS_W // 4
    handles = [
        pltpu.async_copy(x_hbm.at[idx_v.at[i]], rows_v.at[pl.ds(i * q, q)], sem)
        for i in range(4)
    ]
    for h in handles:
        h.wait()
    pltpu.sync_copy(rows_v, xs_hbm.at[pl.ds(wid * ROWS_W, ROWS_W)])


def _project_body(xs_ref, w_sage_ref, w_cls_ref, z4_ref):
    w_fused = jnp.dot(w_sage_ref[...], w_cls_ref[...],
                      preferred_element_type=jnp.float32)
    z = jnp.dot(xs_ref[...], w_fused, preferred_element_type=jnp.float32)
    z4_ref[:, 0:N_CLS] = z
    z4_ref[:, N_CLS:4] = jnp.ones((z.shape[0], 1), jnp.float32)
    z4_ref[:, 4:8] = jnp.zeros((z.shape[0], 4), jnp.float32)


def _project(xs, w_sage, w_cls):
    return pl.pallas_call(
        _project_body,
        out_shape=jax.ShapeDtypeStruct((N_PAD, 8), jnp.float32),
    )(xs, w_sage, w_cls)


@functools.partial(
    pl.kernel,
    out_type=jax.ShapeDtypeStruct((NC, N_PAD, 8), jnp.float32),
    mesh=_sc_mesh,
    scratch_types=[
        pltpu.VMEM((CHUNKS_W, CHUNK), jnp.int32),
        pltpu.VMEM((CHUNKS_W, CHUNK), jnp.int32),
        pltpu.VMEM((EDGES_W, 8), jnp.float32),
        pltpu.VMEM_SHARED((N_PAD, 8), jnp.float32),
        pltpu.SemaphoreType.DMA((CHUNKS_W,)),
        pltpu.SemaphoreType.DMA,
    ],
    compiler_params=pltpu.CompilerParams(use_tc_tiling_on_sc=False),
)
def _edge_aggregate(src_hbm, dst_hbm, zeros_hbm, z4_hbm, acc_hbm,
                    src_v, dst_v, upd_v, acc_sh, gsems, ssem):
    cid = lax.axis_index("c")
    sid = lax.axis_index("s")
    wid = cid * NS + sid

    pltpu.sync_copy(zeros_hbm.at[pl.ds(sid * ROWS_OUT, ROWS_OUT)],
                    acc_sh.at[pl.ds(sid * ROWS_OUT, ROWS_OUT)])
    pltpu.sync_copy(src_hbm.at[wid], src_v)
    pltpu.sync_copy(dst_hbm.at[wid], dst_v)
    plsc.subcore_barrier()

    for j in range(CHUNKS_W):
        pltpu.async_copy(z4_hbm.at[src_v.at[j]],
                         upd_v.at[pl.ds(j * CHUNK, CHUNK)], gsems.at[j])
    for j in range(CHUNKS_W):
        pltpu.make_async_copy(z4_hbm.at[src_v.at[j]],
                              upd_v.at[pl.ds(j * CHUNK, CHUNK)],
                              gsems.at[j]).wait()
        pltpu.async_copy(upd_v.at[pl.ds(j * CHUNK, CHUNK)],
                         acc_sh.at[dst_v.at[j]], ssem, add=True)
    for j in range(CHUNKS_W):
        pltpu.make_async_copy(upd_v.at[pl.ds(j * CHUNK, CHUNK)],
                              acc_sh.at[dst_v.at[j]], ssem).wait()

    plsc.subcore_barrier()
    pltpu.sync_copy(acc_sh.at[pl.ds(sid * ROWS_OUT, ROWS_OUT)],
                    acc_hbm.at[cid].at[pl.ds(sid * ROWS_OUT, ROWS_OUT)])


def _finalize_body(acc_ref, w_cls_ref, b_sage_ref, b_cls_ref, out_ref):
    acc = acc_ref[0] + acc_ref[1]
    cnt = jnp.clip(acc[:N_NODES, 3:4], 1.0, None)
    b_eff = jnp.dot(b_sage_ref[...].reshape(1, -1), w_cls_ref[...],
                    preferred_element_type=jnp.float32) + b_cls_ref[...].reshape(1, -1)
    s = acc[:N_NODES, 0:N_CLS] / cnt + b_eff
    m = jnp.max(s, axis=1, keepdims=True)
    lse = jnp.log(jnp.sum(jnp.exp(s - m), axis=1, keepdims=True)) + m
    out_ref[...] = s - lse


def _finalize(acc, w_cls, b_sage, b_cls):
    return pl.pallas_call(
        _finalize_body,
        out_shape=jax.ShapeDtypeStruct((N_NODES, N_CLS), jnp.float32),
    )(acc, w_cls, b_sage, b_cls)


def kernel(x, n_id, edge_index, W_sage, b_sage, W_cls, b_cls):
    E = edge_index.shape[1]
    nid_pad = jnp.concatenate(
        [n_id, jnp.zeros((N_PAD - N_NODES,), jnp.int32)]).reshape(NW, 4, ROWS_W // 4)
    src_pad = jnp.concatenate(
        [edge_index[0], jnp.zeros((E_PAD - E,), jnp.int32)]).reshape(NW, CHUNKS_W, CHUNK)
    dst_pad = jnp.concatenate(
        [edge_index[1], jnp.full((E_PAD - E,), N_PAD - 1, jnp.int32)]).reshape(NW, CHUNKS_W, CHUNK)
    zeros_init = jnp.zeros((N_PAD, 8), jnp.float32)

    xs = _gather_rows(x, nid_pad)
    z4 = _project(xs, W_sage, W_cls)
    acc = _edge_aggregate(src_pad, dst_pad, zeros_init, z4)
    return _finalize(acc, W_cls, b_sage, b_cls)
